# baseline (speedup 1.0000x reference)
import functools

import jax
import jax.numpy as jnp
import numpy as np
from jax import lax
from jax.experimental import pallas as pl
from jax.experimental.pallas import tpu as pltpu
from jax.experimental.pallas import tpu_sc as plsc

H = 128
NCELL = 12800
NCLU = 1408
STEPS = 4
NW = 32

E_CON, E_MAY, E_LT, E_GT, E_DIFF, E_CLT, E_CGT = (
    12800, 140800, 2560, 2560, 230400, 1280, 1280)

TBL = 12800
(S_A_CON, S_A_MAY, S_A_CLT, S_B_CLT, S_A_CGT, S_B_CGT,
 S_A_LT, S_A_GT, S_A_DIFF, S_B_CON, S_B_MAY, S_B_LT, S_B_GT, S_B_DIFF) = range(14)
NTBL = 14

GO_CON_S = 0
GO_CON_D = GO_CON_S + E_CON
GO_MAY_S = GO_CON_D + E_CON
GO_MAY_D = GO_MAY_S + E_MAY
GO_LT_S = GO_MAY_D + E_MAY
GO_LT_D = GO_LT_S + E_LT
GO_GT_S = GO_LT_D + E_LT
GO_GT_D = GO_GT_S + E_GT
GO_DIFF_S = GO_GT_D + E_GT
GO_DIFF_D = GO_DIFF_S + E_DIFF
GO_CLT_S = GO_DIFF_D + E_DIFF
GO_CLT_D = GO_CLT_S + E_CLT
GO_CGT_S = GO_CLT_D + E_CLT
GO_CGT_D = GO_CGT_S + E_CGT
NG = GO_CGT_D + E_CGT
NGP = 786432
G_CHUNKS = 192
GB_CH = 3
GB_ROWS = GB_CH * 128
G_ROUNDS = G_CHUNKS // GB_CH

MO_CON = 0
MO_MAY = MO_CON + E_CON
MO_LT = MO_MAY + E_MAY
MO_GT = MO_LT + E_LT
MO_DIFF = MO_GT + E_GT
MO_CLT = MO_DIFF + E_DIFF
MO_CGT = MO_CLT + E_CLT
NM = MO_CGT + E_CGT

BLK = 256
SEG_START = [0, 50, 600, 610, 620, 1520, 1525]
SEG_END = [50, 600, 610, 620, 1520, 1525, 1530]
A_BASE = [GO_CON_S // BLK, GO_MAY_S // BLK, GO_LT_S // BLK, GO_GT_S // BLK,
          GO_DIFF_S // BLK, GO_CLT_S // BLK, GO_CGT_S // BLK]
B_BASE = [GO_CON_D // BLK, GO_MAY_D // BLK, GO_LT_D // BLK, GO_GT_D // BLK,
          GO_DIFF_D // BLK, GO_CLT_D // BLK, GO_CGT_D // BLK]
SEG_TYPE = [0, 1, 2, 3, 4, 2, 3]
NBLOCKS = 1530

F32 = jnp.float32


def _seg_select(i, vals):
    out = jnp.int32(vals[6])
    for k in range(5, -1, -1):
        out = jnp.where(i < SEG_END[k], jnp.int32(vals[k]), out)
    return out


def _a_map(i, _=None):
    start = _seg_select(i, SEG_START)
    base = _seg_select(i, A_BASE)
    return base + (i - start), 0


def _b_map(i, _=None):
    start = _seg_select(i, SEG_START)
    base = _seg_select(i, B_BASE)
    return base + (i - start), 0


def _t_map(i, _=None):
    return _seg_select(i, SEG_TYPE), 0, 0


def _clup_body(hclu_ref, p_ref, out_ref):
    out_ref[...] = jnp.tanh(hclu_ref[...] + p_ref[0] + p_ref[1])


_clup = pl.pallas_call(
    _clup_body,
    out_shape=jax.ShapeDtypeStruct((NCLU, H), F32),
)


def _pre_body(hclu_ref, hcell_ref, w_ref, out_ref):
    j = pl.program_id(0)
    w = w_ref[0]

    @pl.when(j < 6)
    def _():
        out_ref[0:NCLU, :] = jnp.dot(hclu_ref[...], w,
                                     preferred_element_type=F32)

    @pl.when(j >= 6)
    def _():
        out_ref[...] = jnp.dot(hcell_ref[...], w, preferred_element_type=F32)


_pre = pl.pallas_call(
    _pre_body,
    grid=(NTBL,),
    in_specs=[
        pl.BlockSpec((NCLU, H), lambda j: (0, 0)),
        pl.BlockSpec((NCELL, H), lambda j: (0, 0)),
        pl.BlockSpec((1, H, H), lambda j: (j, 0, 0)),
    ],
    out_specs=pl.BlockSpec((TBL, H), lambda j: (j, 0)),
    out_shape=jax.ShapeDtypeStruct((NTBL * TBL, H), F32),
)


BF16 = jnp.bfloat16


def _mlp_body(ga_ref, gb_ref, w2_ref, w3_ref, w4_ref, bs_ref, out_ref):
    b1 = bs_ref[0, 0, :]
    b2 = bs_ref[0, 1, :]
    b3 = bs_ref[0, 2, :]
    b4 = bs_ref[0, 3, :]
    y = jnp.maximum(ga_ref[...] + gb_ref[...] + b1, 0.0).astype(BF16)
    y = jnp.maximum(jnp.dot(y, w2_ref[0], preferred_element_type=F32) + b2,
                    0.0).astype(BF16)
    y = jnp.maximum(jnp.dot(y, w3_ref[0], preferred_element_type=F32) + b3,
                    0.0).astype(BF16)
    out_ref[...] = jnp.dot(y, w4_ref[0], preferred_element_type=F32) + b4


_mlp = pl.pallas_call(
    _mlp_body,
    grid=(NBLOCKS,),
    in_specs=[
        pl.BlockSpec((BLK, H), _a_map),
        pl.BlockSpec((BLK, H), _b_map),
        pl.BlockSpec((1, H, H), _t_map),
        pl.BlockSpec((1, H, H), _t_map),
        pl.BlockSpec((1, H, H), _t_map),
        pl.BlockSpec((1, 4, H), _t_map),
    ],
    out_specs=pl.BlockSpec((BLK, H), lambda i: (i, 0)),
    out_shape=jax.ShapeDtypeStruct((NM, H), F32),
)


def _lstm_factory(first):
    def body(cx_ref, pc_ref, wih_ref, whh_ref, hp_ref, cp_ref, h_ref, c_ref):
        parts = [cx_ref[...]]
        for t in range(5):
            parts.append(pc_ref[t, 0] + pc_ref[t, 1])
        x6 = jnp.concatenate(parts, axis=-1)
        gates = jnp.dot(x6, wih_ref[...], preferred_element_type=F32)
        if not first:
            gates = gates + jnp.dot(hp_ref[...], whh_ref[...],
                                    preferred_element_type=F32)
        i = jax.nn.sigmoid(gates[:, 0:H])
        f = jax.nn.sigmoid(gates[:, H:2 * H])
        g = jnp.tanh(gates[:, 2 * H:3 * H])
        o = jax.nn.sigmoid(gates[:, 3 * H:4 * H])
        cn = i * g if first else f * cp_ref[...] + i * g
        c_ref[...] = cn
        h_ref[...] = o * jnp.tanh(cn)

    nb = NCELL // BLK
    return pl.pallas_call(
        body,
        grid=(nb,),
        in_specs=[
            pl.BlockSpec((BLK, H), lambda b: (b, 0)),
            pl.BlockSpec((5, 2, BLK, H), lambda b: (0, 0, b, 0)),
            pl.BlockSpec((6 * H, 4 * H), lambda b: (0, 0)),
            pl.BlockSpec((H, 4 * H), lambda b: (0, 0)),
            pl.BlockSpec((BLK, H), lambda b: (b, 0)),
            pl.BlockSpec((BLK, H), lambda b: (b, 0)),
        ],
        out_specs=[
            pl.BlockSpec((BLK, H), lambda b: (b, 0)),
            pl.BlockSpec((BLK, H), lambda b: (b, 0)),
        ],
        out_shape=[
            jax.ShapeDtypeStruct((NCELL, H), F32),
            jax.ShapeDtypeStruct((NCELL, H), F32),
        ],
    )


_lstm_first = _lstm_factory(True)
_lstm_rest = _lstm_factory(False)

NPAD = 16384


def _logits_body(h0_ref, h1_ref, h2_ref, h3_ref, oe_ref, out_ref):
    hs = [h0_ref, h1_ref, h2_ref, h3_ref]
    for s in range(4):
        hv = hs[s][...]
        for b in range(8):
            x = hv[128 * b:128 * (b + 1), :]
            out_ref[s, 128 * b:128 * (b + 1), :] = jnp.dot(
                x, oe_ref[b], preferred_element_type=F32)


_logits = pl.pallas_call(
    _logits_body,
    grid=(16,),
    in_specs=[pl.BlockSpec((1024, H), lambda g: (g, 0))] * 4 + [
        pl.BlockSpec((8, H, 16), lambda g: (g, 0, 0)),
    ],
    out_specs=pl.BlockSpec((4, 1024, 16), lambda g: (0, g, 0)),
    out_shape=jax.ShapeDtypeStruct((4, NPAD, 16), F32),
)


@functools.cache
def _sc_gather_kernel():
    mesh = plsc.VectorSubcoreMesh(core_axis_name="c", subcore_axis_name="s")
    return pl.kernel(
        _sc_gather_body,
        out_type=jax.ShapeDtypeStruct((NGP, H), F32),
        mesh=mesh,
        scratch_types=[
            pltpu.VMEM((G_CHUNKS, 128), jnp.int32),
            pltpu.VMEM((GB_ROWS, H), F32),
            pltpu.VMEM((GB_ROWS, H), F32),
            pltpu.SemaphoreType.DMA,
            pltpu.SemaphoreType.DMA,
            pltpu.SemaphoreType.DMA,
            pltpu.SemaphoreType.DMA,
        ],
    )


def _sc_gather(tbl, idx):
    return _sc_gather_kernel()(tbl, idx)


def _sc_gather_body(tbl, idx, out, idx_v, buf0, buf1, gs0, gs1, ws0, ws1):
    c = lax.axis_index("c")
    s = lax.axis_index("s")
    wid = c * 16 + s
    pltpu.sync_copy(idx.at[wid], idx_v)
    base = wid * G_CHUNKS * 128

    def fire(r, buf, sem):
        for k in range(GB_CH):
            pltpu.async_copy(tbl.at[idx_v.at[r * GB_CH + k]],
                             buf.at[pl.ds(k * 128, 128)], sem)

    def drain_gathers(buf, sem):
        pltpu.make_async_copy(tbl.at[pl.ds(0, GB_ROWS)], buf, sem).wait()

    def out_rows(r):
        return out.at[pl.ds(base + r * GB_ROWS, GB_ROWS)]

    fire(0, buf0, gs0)
    fire(1, buf1, gs1)

    def body(it, _):
        r0 = it * 2
        r1 = r0 + 1
        drain_gathers(buf0, gs0)
        pltpu.async_copy(buf0, out_rows(r0), ws0)
        drain_gathers(buf1, gs1)
        pltpu.async_copy(buf1, out_rows(r1), ws1)

        @pl.when(it < G_ROUNDS // 2 - 1)
        def _():
            pltpu.make_async_copy(buf0, out_rows(r0), ws0).wait()
            fire(r0 + 2, buf0, gs0)
            pltpu.make_async_copy(buf1, out_rows(r1), ws1).wait()
            fire(r1 + 2, buf1, gs1)
        return 0

    lax.fori_loop(0, G_ROUNDS // 2, body, 0)
    last = G_ROUNDS - 1
    pltpu.make_async_copy(buf0, out_rows(last - 1), ws0).wait()
    pltpu.make_async_copy(buf1, out_rows(last), ws1).wait()


CSZ = 80


@functools.cache
def _sc_scatter_kernel():
    mesh = plsc.VectorSubcoreMesh(core_axis_name="c", subcore_axis_name="s")
    return pl.kernel(
        _sc_scatter_body,
        out_type=(
            jax.ShapeDtypeStruct((5, 2, NCELL, H), F32),
            jax.ShapeDtypeStruct((2, NCLU, H), F32),
        ),
        mesh=mesh,
        scratch_types=[
            pltpu.VMEM_SHARED((NCELL, H), F32),
            pltpu.VMEM((2, CSZ), jnp.int32),
            pltpu.VMEM((CSZ, H), F32),
            pltpu.VMEM((CSZ, H), F32),
            pltpu.SemaphoreType.DMA,
            pltpu.SemaphoreType.DMA,
        ],
    )


def _sc_scatter(m, ixcon, ixmay, ixlt, ixgt, ixdiff, ixclu, zz):
    return _sc_scatter_kernel()(m, ixcon, ixmay, ixlt, ixgt, ixdiff, ixclu, zz)


def _sc_scatter_body(m, ixcon, ixmay, ixlt, ixgt, ixdiff, ixclu, zz,
                     pcell, pclu, shc, ix2, mb0, mb1, sm0, sm1):
    c = lax.axis_index("c")
    s = lax.axis_index("s")
    wid = c * 16 + s
    for k in range(4):
        pltpu.sync_copy(zz, shc.at[pl.ds(s * 800 + k * 200, 200)])

    def cell_phase(t, ix, nch, mbase):
        tb = mbase + wid * nch * CSZ
        ixw = ix.at[wid]
        bufs = (mb0, mb1)
        sems = (sm0, sm1)

        def issue(j, b):
            pltpu.async_copy(m.at[pl.ds(tb + j * CSZ, CSZ)], bufs[b], sems[b])
            pltpu.async_copy(ixw.at[j], ix2.at[b], sems[b])

        def wait_add(j, b):
            pltpu.make_async_copy(m.at[pl.ds(tb + j * CSZ, CSZ)],
                                  bufs[b], sems[b]).wait()
            pltpu.make_async_copy(ixw.at[j], ix2.at[b], sems[b]).wait()
            pltpu.sync_copy(bufs[b], shc.at[ix2.at[b]], add=True)

        issue(0, 0)
        if nch > 1:
            issue(1, 1)
        plsc.subcore_barrier()

        if nch <= 2:
            wait_add(0, 0)
            if nch == 2:
                wait_add(1, 1)
        else:
            def loop(it, carry):
                j0 = it * 2
                j1 = j0 + 1
                wait_add(j0, 0)

                @pl.when(j0 + 2 < nch)
                def _():
                    issue(j0 + 2, 0)

                @pl.when(j1 < nch)
                def _():
                    wait_add(j1, 1)

                    @pl.when(j1 + 2 < nch)
                    def _():
                        issue(j1 + 2, 1)
                return carry

            lax.fori_loop(0, (nch + 1) // 2, loop, 0)
        plsc.subcore_barrier()
        for k in range(4):
            rows = pl.ds(s * 800 + k * 200, 200)
            pltpu.sync_copy(shc.at[rows], pcell.at[t].at[c].at[rows])
            pltpu.sync_copy(zz, shc.at[rows])

    cell_phase(0, ixcon, 5, MO_CON)
    cell_phase(1, ixmay, 55, MO_MAY)
    cell_phase(2, ixlt, 1, MO_LT)
    cell_phase(3, ixgt, 1, MO_GT)
    cell_phase(4, ixdiff, 90, MO_DIFF)

    pltpu.sync_copy(ixclu.at[wid], ix2.at[pl.ds(0, 1)])
    base = jnp.where(wid < 16, MO_CLT + wid * CSZ, MO_CGT + (wid - 16) * CSZ)
    plsc.subcore_barrier()
    pltpu.sync_copy(m.at[pl.ds(base, CSZ)], mb0)
    pltpu.sync_copy(mb0, shc.at[ix2.at[0]], add=True)
    plsc.subcore_barrier()
    rows = pl.ds(s * 88, 88)
    pltpu.sync_copy(shc.at[rows], pclu.at[c].at[rows])


def kernel(cell_x, cluster_x, output_embeddings, params, contains_src,
           contains_dst, may_src, may_dst, lt_edges, gt_edges, diff_edges,
           clt_edges, cgt_edges):
    i32 = jnp.int32
    gidx = jnp.concatenate([
        contains_src.astype(i32) + S_A_CON * TBL,
        contains_dst.astype(i32) + S_B_CON * TBL,
        may_src.astype(i32) + S_A_MAY * TBL,
        may_dst.astype(i32) + S_B_MAY * TBL,
        lt_edges[0].astype(i32) + S_A_LT * TBL,
        lt_edges[1].astype(i32) + S_B_LT * TBL,
        gt_edges[0].astype(i32) + S_A_GT * TBL,
        gt_edges[1].astype(i32) + S_B_GT * TBL,
        diff_edges[0].astype(i32) + S_A_DIFF * TBL,
        diff_edges[1].astype(i32) + S_B_DIFF * TBL,
        clt_edges[0].astype(i32) + S_A_CLT * TBL,
        clt_edges[1].astype(i32) + S_B_CLT * TBL,
        cgt_edges[0].astype(i32) + S_A_CGT * TBL,
        cgt_edges[1].astype(i32) + S_B_CGT * TBL,
        jnp.zeros((NGP - NG,), i32),
    ]).reshape(NW, G_CHUNKS, 128)

    ixcon = contains_dst.astype(i32).reshape(NW, 5, CSZ)
    ixmay = may_dst.astype(i32).reshape(NW, 55, CSZ)
    ixlt = lt_edges[1].astype(i32).reshape(NW, 1, CSZ)
    ixgt = gt_edges[1].astype(i32).reshape(NW, 1, CSZ)
    ixdiff = diff_edges[1].astype(i32).reshape(NW, 90, CSZ)
    ixclu = jnp.concatenate(
        [clt_edges[1].astype(i32), cgt_edges[1].astype(i32)]
    ).reshape(NW, 1, CSZ)
    zz = jnp.zeros((200, H), F32)

    order = ['contains', 'may_contain', 'lt', 'gt', 'diff']
    w1a = {t: params[t]['W1'][:H] for t in order}
    w1b = {t: params[t]['W1'][H:] for t in order}
    w14 = jnp.stack([
        w1a['contains'], w1a['may_contain'],
        w1a['lt'], w1b['lt'], w1a['gt'], w1b['gt'],
        w1a['lt'], w1a['gt'], w1a['diff'],
        w1b['contains'], w1b['may_contain'], w1b['lt'], w1b['gt'], w1b['diff'],
    ])
    bf16 = jnp.bfloat16
    w2s = jnp.stack([params[t]['W2'] for t in order]).astype(bf16)
    w3s = jnp.stack([params[t]['W3'] for t in order]).astype(bf16)
    w4s = jnp.stack([params[t]['W4'] for t in order]).astype(bf16)
    bs = jnp.stack([
        jnp.stack([params[t]['b1'], params[t]['b2'],
                   params[t]['b3'], params[t]['b4']]) for t in order])
    wihT = params['Wih'].T
    whhT = params['Whh'].T

    oeT = jnp.pad(
        output_embeddings.reshape(128, 11, H).transpose(0, 2, 1),
        ((0, 0), (0, 0), (0, 5)))

    h_cell = cell_x
    h_clu = cluster_x
    rnn_h = rnn_c = None
    p_clu = None
    outs = []
    for step in range(STEPS):
        if step:
            h_clu = _clup(h_clu, p_clu)
        tblv = _pre(h_clu, h_cell, w14)
        g = _sc_gather(tblv, gidx)
        msg = _mlp(g, g, w2s, w3s, w4s, bs)
        p_cell, p_clu = _sc_scatter(msg, ixcon, ixmay, ixlt, ixgt, ixdiff,
                                    ixclu, zz)
        if step == 0:
            zeros = jnp.zeros((NCELL, H), F32)
            rnn_h, rnn_c = _lstm_first(cell_x, p_cell, wihT, whhT,
                                       zeros, zeros)
        else:
            rnn_h, rnn_c = _lstm_rest(cell_x, p_cell, wihT, whhT,
                                      rnn_h, rnn_c)
        h_cell = rnn_h
        outs.append(rnn_h)

    hp = [
        jnp.pad(h.reshape(128, 100, H), ((0, 0), (0, 28), (0, 0)))
        .reshape(NPAD, H) for h in outs
    ]
    lg = _logits(hp[0], hp[1], hp[2], hp[3], oeT)
    return lg.reshape(4, 128, 128, 16)[:, :, :100, :11].reshape(4, NCELL, 11)


# speedup vs baseline: 1.4879x; 1.4879x over previous
import functools

import jax
import jax.numpy as jnp
import numpy as np
from jax import lax
from jax.experimental import pallas as pl
from jax.experimental.pallas import tpu as pltpu
from jax.experimental.pallas import tpu_sc as plsc

H = 128
NCELL = 12800
NCLU = 1408
STEPS = 4
NW = 32

E_CON, E_MAY, E_LT, E_GT, E_DIFF, E_CLT, E_CGT = (
    12800, 140800, 2560, 2560, 230400, 1280, 1280)
E_CLU_PAD = 1536

TBL = 12800
(S_A_CON, S_A_MAY, S_A_CLT, S_B_CLT, S_A_CGT, S_B_CGT,
 S_A_LT, S_A_GT, S_A_DIFF, S_B_CON, S_B_MAY, S_B_LT, S_B_GT, S_B_DIFF) = range(14)
NTBL = 14

GO_CON_S = 0
GO_CON_D = GO_CON_S + E_CON
GO_MAY_S = GO_CON_D + E_CON
GO_MAY_D = GO_MAY_S + E_MAY
GO_LT_S = GO_MAY_D + E_MAY
GO_LT_D = GO_LT_S + E_LT
GO_GT_S = GO_LT_D + E_LT
GO_GT_D = GO_GT_S + E_GT
NG1 = GO_GT_D + E_GT
GCH1 = 78
NGP1 = NW * GCH1 * 128
MO_CON = 0
MO_MAY = MO_CON + E_CON
MO_LT = MO_MAY + E_MAY
MO_GT = MO_LT + E_LT
NM1 = MO_GT + E_GT

GO_DIFF_S = 0
GO_DIFF_D = GO_DIFF_S + E_DIFF
GO_CLT_S = GO_DIFF_D + E_DIFF
GO_CLT_D = GO_CLT_S + E_CLU_PAD
GO_CGT_S = GO_CLT_D + E_CLU_PAD
GO_CGT_D = GO_CGT_S + E_CLU_PAD
NG2 = GO_CGT_D + E_CLU_PAD
GCH2 = 114
NGP2 = NW * GCH2 * 128
MO_DIFF = 0
MO_CLT = MO_DIFF + E_DIFF
MO_CGT = MO_CLT + E_CLU_PAD
NM2 = MO_CGT + E_CLU_PAD

BLK = 512
SEGS1 = dict(
    ends=(25, 300, 305, 310),
    starts=(0, 25, 300, 305),
    a=(GO_CON_S // BLK, GO_MAY_S // BLK, GO_LT_S // BLK, GO_GT_S // BLK),
    b=(GO_CON_D // BLK, GO_MAY_D // BLK, GO_LT_D // BLK, GO_GT_D // BLK),
    types=(0, 1, 2, 3), nblocks=310, nm=NM1)
SEGS2 = dict(
    ends=(450, 453, 456),
    starts=(0, 450, 453),
    a=(GO_DIFF_S // BLK, GO_CLT_S // BLK, GO_CGT_S // BLK),
    b=(GO_DIFF_D // BLK, GO_CLT_D // BLK, GO_CGT_D // BLK),
    types=(4, 2, 3), nblocks=456, nm=NM2)

GB_CH = 3
GB_ROWS = GB_CH * 128

F32 = jnp.float32
BF16 = jnp.bfloat16


def _clup_body(hclu_ref, p_ref, out_ref):
    out_ref[...] = jnp.tanh(hclu_ref[...] + p_ref[0] + p_ref[1])


_clup = pl.pallas_call(
    _clup_body,
    out_shape=jax.ShapeDtypeStruct((NCLU, H), F32),
)


def _pre_body(hclu_ref, hcell_ref, w_ref, out_ref):
    j = pl.program_id(0)
    w = w_ref[0]

    @pl.when(j < 6)
    def _():
        out_ref[0:NCLU, :] = jnp.dot(hclu_ref[...], w,
                                     preferred_element_type=F32)

    @pl.when(j >= 6)
    def _():
        out_ref[...] = jnp.dot(hcell_ref[...], w, preferred_element_type=F32)


_pre = pl.pallas_call(
    _pre_body,
    grid=(NTBL,),
    in_specs=[
        pl.BlockSpec((NCLU, H), lambda j: (0, 0)),
        pl.BlockSpec((NCELL, H), lambda j: (0, 0)),
        pl.BlockSpec((1, H, H), lambda j: (j, 0, 0)),
    ],
    out_specs=pl.BlockSpec((TBL, H), lambda j: (j, 0)),
    out_shape=jax.ShapeDtypeStruct((NTBL * TBL, H), F32),
)


def _mlp_body(ga_ref, gb_ref, w2_ref, w3_ref, w4_ref, bs_ref, out_ref):
    b1 = bs_ref[0, 0, :]
    b2 = bs_ref[0, 1, :]
    b3 = bs_ref[0, 2, :]
    b4 = bs_ref[0, 3, :]
    y = jnp.maximum(ga_ref[...] + gb_ref[...] + b1, 0.0).astype(BF16)
    y = jnp.maximum(jnp.dot(y, w2_ref[0], preferred_element_type=F32) + b2,
                    0.0).astype(BF16)
    y = jnp.maximum(jnp.dot(y, w3_ref[0], preferred_element_type=F32) + b3,
                    0.0).astype(BF16)
    out_ref[...] = jnp.dot(y, w4_ref[0], preferred_element_type=F32) + b4


def _mk_mlp(segs):
    ends, starts = segs['ends'], segs['starts']
    abase, bbase, types = segs['a'], segs['b'], segs['types']
    n = len(ends)

    def sel(i, vals):
        out = jnp.int32(vals[n - 1])
        for k in range(n - 2, -1, -1):
            out = jnp.where(i < ends[k], jnp.int32(vals[k]), out)
        return out

    def a_map(i):
        return sel(i, abase) + (i - sel(i, starts)), 0

    def b_map(i):
        return sel(i, bbase) + (i - sel(i, starts)), 0

    def t_map(i):
        return sel(i, types), 0, 0

    return pl.pallas_call(
        _mlp_body,
        grid=(segs['nblocks'],),
        in_specs=[
            pl.BlockSpec((BLK, H), a_map),
            pl.BlockSpec((BLK, H), b_map),
            pl.BlockSpec((1, H, H), t_map),
            pl.BlockSpec((1, H, H), t_map),
            pl.BlockSpec((1, H, H), t_map),
            pl.BlockSpec((1, 4, H), t_map),
        ],
        out_specs=pl.BlockSpec((BLK, H), lambda i: (i, 0)),
        out_shape=jax.ShapeDtypeStruct((segs['nm'], H), F32),
    )


_mlp1 = _mk_mlp(SEGS1)
_mlp2 = _mk_mlp(SEGS2)


def _lstm_factory(first):
    def body(cx_ref, pc_ref, p5_ref, wih_ref, whh_ref, hp_ref, cp_ref,
             h_ref, c_ref):
        parts = [cx_ref[...]]
        for t in range(4):
            parts.append(pc_ref[t, 0] + pc_ref[t, 1])
        parts.append(p5_ref[0] + p5_ref[1])
        x6 = jnp.concatenate(parts, axis=-1)
        gates = jnp.dot(x6, wih_ref[...], preferred_element_type=F32)
        if not first:
            gates = gates + jnp.dot(hp_ref[...], whh_ref[...],
                                    preferred_element_type=F32)
        i = jax.nn.sigmoid(gates[:, 0:H])
        f = jax.nn.sigmoid(gates[:, H:2 * H])
        g = jnp.tanh(gates[:, 2 * H:3 * H])
        o = jax.nn.sigmoid(gates[:, 3 * H:4 * H])
        cn = i * g if first else f * cp_ref[...] + i * g
        c_ref[...] = cn
        h_ref[...] = o * jnp.tanh(cn)

    nb = NCELL // BLK
    return pl.pallas_call(
        body,
        grid=(nb,),
        in_specs=[
            pl.BlockSpec((BLK, H), lambda b: (b, 0)),
            pl.BlockSpec((4, 2, BLK, H), lambda b: (0, 0, b, 0)),
            pl.BlockSpec((2, BLK, H), lambda b: (0, b, 0)),
            pl.BlockSpec((6 * H, 4 * H), lambda b: (0, 0)),
            pl.BlockSpec((H, 4 * H), lambda b: (0, 0)),
            pl.BlockSpec((BLK, H), lambda b: (b, 0)),
            pl.BlockSpec((BLK, H), lambda b: (b, 0)),
        ],
        out_specs=[
            pl.BlockSpec((BLK, H), lambda b: (b, 0)),
            pl.BlockSpec((BLK, H), lambda b: (b, 0)),
        ],
        out_shape=[
            jax.ShapeDtypeStruct((NCELL, H), F32),
            jax.ShapeDtypeStruct((NCELL, H), F32),
        ],
    )


_lstm_first = _lstm_factory(True)
_lstm_rest = _lstm_factory(False)

NPAD = 16384


def _logits_body(h0_ref, h1_ref, h2_ref, h3_ref, oe_ref, out_ref):
    hs = [h0_ref, h1_ref, h2_ref, h3_ref]
    for s in range(4):
        hv = hs[s][...]
        for b in range(8):
            x = hv[128 * b:128 * (b + 1), :]
            out_ref[s, 128 * b:128 * (b + 1), :] = jnp.dot(
                x, oe_ref[b], preferred_element_type=F32)


_logits = pl.pallas_call(
    _logits_body,
    grid=(16,),
    in_specs=[pl.BlockSpec((1024, H), lambda g: (g, 0))] * 4 + [
        pl.BlockSpec((8, H, 16), lambda g: (g, 0, 0)),
    ],
    out_specs=pl.BlockSpec((4, 1024, 16), lambda g: (0, g, 0)),
    out_shape=jax.ShapeDtypeStruct((4, NPAD, 16), F32),
)


def _mk_gather_body(gch):
    nrounds = gch // GB_CH

    def body(tbl, idx, out, idx_v, buf0, buf1, gs0, gs1, ws0, ws1):
        c = lax.axis_index("c")
        s = lax.axis_index("s")
        wid = c * 16 + s
        pltpu.sync_copy(idx.at[wid], idx_v)
        base = wid * gch * 128

        def fire(r, buf, sem):
            for k in range(GB_CH):
                pltpu.async_copy(tbl.at[idx_v.at[r * GB_CH + k]],
                                 buf.at[pl.ds(k * 128, 128)], sem)

        def drain_gathers(buf, sem):
            pltpu.make_async_copy(tbl.at[pl.ds(0, GB_ROWS)], buf, sem).wait()

        def out_rows(r):
            return out.at[pl.ds(base + r * GB_ROWS, GB_ROWS)]

        fire(0, buf0, gs0)
        fire(1, buf1, gs1)

        def loop(it, _):
            r0 = it * 2
            r1 = r0 + 1
            drain_gathers(buf0, gs0)
            pltpu.async_copy(buf0, out_rows(r0), ws0)
            drain_gathers(buf1, gs1)
            pltpu.async_copy(buf1, out_rows(r1), ws1)

            @pl.when(it < nrounds // 2 - 1)
            def _():
                pltpu.make_async_copy(buf0, out_rows(r0), ws0).wait()
                fire(r0 + 2, buf0, gs0)
                pltpu.make_async_copy(buf1, out_rows(r1), ws1).wait()
                fire(r1 + 2, buf1, gs1)
            return 0

        lax.fori_loop(0, nrounds // 2, loop, 0)
        last = nrounds - 1
        pltpu.make_async_copy(buf0, out_rows(last - 1), ws0).wait()
        pltpu.make_async_copy(buf1, out_rows(last), ws1).wait()

    return body


@functools.cache
def _sc_gather_kernel(gch, nrows):
    mesh = plsc.VectorSubcoreMesh(core_axis_name="c", subcore_axis_name="s")
    return pl.kernel(
        _mk_gather_body(gch),
        out_type=jax.ShapeDtypeStruct((nrows, H), F32),
        mesh=mesh,
        scratch_types=[
            pltpu.VMEM((gch, 128), jnp.int32),
            pltpu.VMEM((GB_ROWS, H), F32),
            pltpu.VMEM((GB_ROWS, H), F32),
            pltpu.SemaphoreType.DMA,
            pltpu.SemaphoreType.DMA,
            pltpu.SemaphoreType.DMA,
            pltpu.SemaphoreType.DMA,
        ],
    )


def _sc_gather1(tbl, idx):
    return _sc_gather_kernel(GCH1, NGP1)(tbl, idx)


def _sc_gather2(tbl, idx):
    return _sc_gather_kernel(GCH2, NGP2)(tbl, idx)


CSZ = 80


def _scatter_cell_phase(m, shc, ix2, mb0, mb1, sm0, sm1, s, wid,
                        pcell_view, zz, ix, nch, mbase):
    tb = mbase + wid * nch * CSZ
    ixw = ix.at[wid]
    bufs = (mb0, mb1)
    sems = (sm0, sm1)

    def issue(j, b):
        pltpu.async_copy(m.at[pl.ds(tb + j * CSZ, CSZ)], bufs[b], sems[b])
        pltpu.async_copy(ixw.at[j], ix2.at[b], sems[b])

    def wait_add(j, b):
        pltpu.make_async_copy(m.at[pl.ds(tb + j * CSZ, CSZ)],
                              bufs[b], sems[b]).wait()
        pltpu.make_async_copy(ixw.at[j], ix2.at[b], sems[b]).wait()
        pltpu.sync_copy(bufs[b], shc.at[ix2.at[b]], add=True)

    issue(0, 0)
    if nch > 1:
        issue(1, 1)
    plsc.subcore_barrier()

    if nch <= 2:
        wait_add(0, 0)
        if nch == 2:
            wait_add(1, 1)
    else:
        def loop(it, carry):
            j0 = it * 2
            j1 = j0 + 1
            wait_add(j0, 0)

            @pl.when(j0 + 2 < nch)
            def _():
                issue(j0 + 2, 0)

            @pl.when(j1 < nch)
            def _():
                wait_add(j1, 1)

                @pl.when(j1 + 2 < nch)
                def _():
                    issue(j1 + 2, 1)
            return carry

        lax.fori_loop(0, (nch + 1) // 2, loop, 0)
    plsc.subcore_barrier()
    for k in range(4):
        rows = pl.ds(s * 800 + k * 200, 200)
        pltpu.sync_copy(shc.at[rows], pcell_view.at[rows])
        pltpu.sync_copy(zz, shc.at[rows])


def _sc_scatter1_body(m, ixcon, ixmay, ixlt, ixgt, zz,
                      pcell, shc, ix2, mb0, mb1, sm0, sm1):
    c = lax.axis_index("c")
    s = lax.axis_index("s")
    wid = c * 16 + s
    for k in range(4):
        pltpu.sync_copy(zz, shc.at[pl.ds(s * 800 + k * 200, 200)])

    args = (m, shc, ix2, mb0, mb1, sm0, sm1, s, wid)
    _scatter_cell_phase(*args, pcell.at[0].at[c], zz, ixcon, 5, MO_CON)
    _scatter_cell_phase(*args, pcell.at[1].at[c], zz, ixmay, 55, MO_MAY)
    _scatter_cell_phase(*args, pcell.at[2].at[c], zz, ixlt, 1, MO_LT)
    _scatter_cell_phase(*args, pcell.at[3].at[c], zz, ixgt, 1, MO_GT)


def _sc_scatter2_body(m, ixdiff, ixclu, zz,
                      pcell, pclu, shc, ix2, mb0, mb1, sm0, sm1):
    c = lax.axis_index("c")
    s = lax.axis_index("s")
    wid = c * 16 + s
    for k in range(4):
        pltpu.sync_copy(zz, shc.at[pl.ds(s * 800 + k * 200, 200)])

    args = (m, shc, ix2, mb0, mb1, sm0, sm1, s, wid)
    _scatter_cell_phase(*args, pcell.at[c], zz, ixdiff, 90, MO_DIFF)

    pltpu.sync_copy(ixclu.at[wid], ix2.at[pl.ds(0, 1)])
    base = jnp.where(wid < 16, MO_CLT + wid * CSZ, MO_CGT + (wid - 16) * CSZ)
    plsc.subcore_barrier()
    pltpu.sync_copy(m.at[pl.ds(base, CSZ)], mb0)
    pltpu.sync_copy(mb0, shc.at[ix2.at[0]], add=True)
    plsc.subcore_barrier()
    rows = pl.ds(s * 88, 88)
    pltpu.sync_copy(shc.at[rows], pclu.at[c].at[rows])


def _sc_scratch():
    return [
        pltpu.VMEM_SHARED((NCELL, H), F32),
        pltpu.VMEM((2, CSZ), jnp.int32),
        pltpu.VMEM((CSZ, H), F32),
        pltpu.VMEM((CSZ, H), F32),
        pltpu.SemaphoreType.DMA,
        pltpu.SemaphoreType.DMA,
    ]


@functools.cache
def _sc_scatter1_kernel():
    mesh = plsc.VectorSubcoreMesh(core_axis_name="c", subcore_axis_name="s")
    return pl.kernel(
        _sc_scatter1_body,
        out_type=jax.ShapeDtypeStruct((4, 2, NCELL, H), F32),
        mesh=mesh,
        scratch_types=_sc_scratch(),
    )


@functools.cache
def _sc_scatter2_kernel():
    mesh = plsc.VectorSubcoreMesh(core_axis_name="c", subcore_axis_name="s")
    return pl.kernel(
        _sc_scatter2_body,
        out_type=(
            jax.ShapeDtypeStruct((2, NCELL, H), F32),
            jax.ShapeDtypeStruct((2, NCLU, H), F32),
        ),
        mesh=mesh,
        scratch_types=_sc_scratch(),
    )


def _sc_scatter1(m, ixcon, ixmay, ixlt, ixgt, zz):
    return _sc_scatter1_kernel()(m, ixcon, ixmay, ixlt, ixgt, zz)


def _sc_scatter2(m, ixdiff, ixclu, zz):
    return _sc_scatter2_kernel()(m, ixdiff, ixclu, zz)


def kernel(cell_x, cluster_x, output_embeddings, params, contains_src,
           contains_dst, may_src, may_dst, lt_edges, gt_edges, diff_edges,
           clt_edges, cgt_edges):
    i32 = jnp.int32
    gidx1 = jnp.concatenate([
        contains_src.astype(i32) + S_A_CON * TBL,
        contains_dst.astype(i32) + S_B_CON * TBL,
        may_src.astype(i32) + S_A_MAY * TBL,
        may_dst.astype(i32) + S_B_MAY * TBL,
        lt_edges[0].astype(i32) + S_A_LT * TBL,
        lt_edges[1].astype(i32) + S_B_LT * TBL,
        gt_edges[0].astype(i32) + S_A_GT * TBL,
        gt_edges[1].astype(i32) + S_B_GT * TBL,
        jnp.zeros((NGP1 - NG1,), i32),
    ]).reshape(NW, GCH1, 128)
    gidx2 = jnp.concatenate([
        diff_edges[0].astype(i32) + S_A_DIFF * TBL,
        diff_edges[1].astype(i32) + S_B_DIFF * TBL,
        jnp.pad(clt_edges[0].astype(i32) + S_A_CLT * TBL, (0, 256)),
        jnp.pad(clt_edges[1].astype(i32) + S_B_CLT * TBL, (0, 256)),
        jnp.pad(cgt_edges[0].astype(i32) + S_A_CGT * TBL, (0, 256)),
        jnp.pad(cgt_edges[1].astype(i32) + S_B_CGT * TBL, (0, 256)),
    ]).reshape(NW, GCH2, 128)

    ixcon = contains_dst.astype(i32).reshape(NW, 5, CSZ)
    ixmay = may_dst.astype(i32).reshape(NW, 55, CSZ)
    ixlt = lt_edges[1].astype(i32).reshape(NW, 1, CSZ)
    ixgt = gt_edges[1].astype(i32).reshape(NW, 1, CSZ)
    ixdiff = diff_edges[1].astype(i32).reshape(NW, 90, CSZ)
    ixclu = jnp.concatenate(
        [clt_edges[1].astype(i32), cgt_edges[1].astype(i32)]
    ).reshape(NW, 1, CSZ)
    zz = jnp.zeros((200, H), F32)

    order = ['contains', 'may_contain', 'lt', 'gt', 'diff']
    w1a = {t: params[t]['W1'][:H] for t in order}
    w1b = {t: params[t]['W1'][H:] for t in order}
    w14 = jnp.stack([
        w1a['contains'], w1a['may_contain'],
        w1a['lt'], w1b['lt'], w1a['gt'], w1b['gt'],
        w1a['lt'], w1a['gt'], w1a['diff'],
        w1b['contains'], w1b['may_contain'], w1b['lt'], w1b['gt'], w1b['diff'],
    ])
    w2s = jnp.stack([params[t]['W2'] for t in order]).astype(BF16)
    w3s = jnp.stack([params[t]['W3'] for t in order]).astype(BF16)
    w4s = jnp.stack([params[t]['W4'] for t in order]).astype(BF16)
    bs = jnp.stack([
        jnp.stack([params[t]['b1'], params[t]['b2'],
                   params[t]['b3'], params[t]['b4']]) for t in order])
    wihT = params['Wih'].T
    whhT = params['Whh'].T

    oeT = jnp.pad(
        output_embeddings.reshape(128, 11, H).transpose(0, 2, 1),
        ((0, 0), (0, 0), (0, 5)))

    h_cell = cell_x
    h_clu = cluster_x
    rnn_h = rnn_c = None
    p_clu = None
    outs = []
    for step in range(STEPS):
        if step:
            h_clu = _clup(h_clu, p_clu)
        tblv = _pre(h_clu, h_cell, w14)
        g1 = _sc_gather1(tblv, gidx1)
        m1 = _mlp1(g1, g1, w2s, w3s, w4s, bs)
        g2 = _sc_gather2(tblv, gidx2)
        m2 = _mlp2(g2, g2, w2s, w3s, w4s, bs)
        pc14 = _sc_scatter1(m1, ixcon, ixmay, ixlt, ixgt, zz)
        pc5, p_clu = _sc_scatter2(m2, ixdiff, ixclu, zz)
        if step == 0:
            zeros = jnp.zeros((NCELL, H), F32)
            rnn_h, rnn_c = _lstm_first(cell_x, pc14, pc5, wihT, whhT,
                                       zeros, zeros)
        else:
            rnn_h, rnn_c = _lstm_rest(cell_x, pc14, pc5, wihT, whhT,
                                      rnn_h, rnn_c)
        h_cell = rnn_h
        outs.append(rnn_h)

    hp = [
        jnp.pad(h.reshape(128, 100, H), ((0, 0), (0, 28), (0, 0)))
        .reshape(NPAD, H) for h in outs
    ]
    lg = _logits(hp[0], hp[1], hp[2], hp[3], oeT)
    return lg.reshape(4, 128, 128, 16)[:, :, :100, :11].reshape(4, NCELL, 11)


# speedup vs baseline: 1.5407x; 1.0354x over previous
import functools

import jax
import jax.numpy as jnp
import numpy as np
from jax import lax
from jax.experimental import pallas as pl
from jax.experimental.pallas import tpu as pltpu
from jax.experimental.pallas import tpu_sc as plsc

H = 128
NCELL = 12800
NCLU = 1408
STEPS = 4
NW = 32

E_CON, E_MAY, E_LT, E_GT, E_DIFF, E_CLT, E_CGT = (
    12800, 140800, 2560, 2560, 230400, 1280, 1280)
E_CLU_PAD = 1536

TBL = 12800
(S_A_CON, S_A_MAY, S_A_CLT, S_B_CLT, S_A_CGT, S_B_CGT,
 S_A_LT, S_A_GT, S_A_DIFF, S_B_CON, S_B_MAY, S_B_LT, S_B_GT, S_B_DIFF) = range(14)
NTBL = 14

GO_CON_S = 0
GO_CON_D = GO_CON_S + E_CON
GO_MAY_S = GO_CON_D + E_CON
GO_MAY_D = GO_MAY_S + E_MAY
GO_LT_S = GO_MAY_D + E_MAY
GO_LT_D = GO_LT_S + E_LT
GO_GT_S = GO_LT_D + E_LT
GO_GT_D = GO_GT_S + E_GT
NG1 = GO_GT_D + E_GT
GCH1 = 78
NGP1 = NW * GCH1 * 128
MO_CON = 0
MO_MAY = MO_CON + E_CON
MO_LT = MO_MAY + E_MAY
MO_GT = MO_LT + E_LT
NM1 = MO_GT + E_GT

GO_DIFF_S = 0
GO_DIFF_D = GO_DIFF_S + E_DIFF
GO_CLT_S = GO_DIFF_D + E_DIFF
GO_CLT_D = GO_CLT_S + E_CLU_PAD
GO_CGT_S = GO_CLT_D + E_CLU_PAD
GO_CGT_D = GO_CGT_S + E_CLU_PAD
NG2 = GO_CGT_D + E_CLU_PAD
GCH2 = 114
NGP2 = NW * GCH2 * 128
MO_DIFF = 0
MO_CLT = MO_DIFF + E_DIFF
MO_CGT = MO_CLT + E_CLU_PAD
NM2 = MO_CGT + E_CLU_PAD

BLK = 512
SEGS1 = dict(
    ends=(25, 300, 305, 310),
    starts=(0, 25, 300, 305),
    a=(GO_CON_S // BLK, GO_MAY_S // BLK, GO_LT_S // BLK, GO_GT_S // BLK),
    b=(GO_CON_D // BLK, GO_MAY_D // BLK, GO_LT_D // BLK, GO_GT_D // BLK),
    types=(0, 1, 2, 3), nblocks=310, nm=NM1)
SEGS2 = dict(
    ends=(450, 453, 456),
    starts=(0, 450, 453),
    a=(GO_DIFF_S // BLK, GO_CLT_S // BLK, GO_CGT_S // BLK),
    b=(GO_DIFF_D // BLK, GO_CLT_D // BLK, GO_CGT_D // BLK),
    types=(4, 2, 3), nblocks=456, nm=NM2)

GB_CH = 3
GB_ROWS = GB_CH * 128

F32 = jnp.float32
BF16 = jnp.bfloat16


def _clup_body(hclu_ref, p_ref, out_ref):
    out_ref[...] = jnp.tanh(hclu_ref[...] + p_ref[0] + p_ref[1])


_clup = pl.pallas_call(
    _clup_body,
    out_shape=jax.ShapeDtypeStruct((NCLU, H), F32),
)


def _pre_body(hclu_ref, hcell_ref, w_ref, out_ref):
    j = pl.program_id(0)
    w = w_ref[0]

    @pl.when(j < 6)
    def _():
        out_ref[0:NCLU, :] = jnp.dot(hclu_ref[...], w,
                                     preferred_element_type=F32)

    @pl.when(j >= 6)
    def _():
        out_ref[...] = jnp.dot(hcell_ref[...], w, preferred_element_type=F32)


_pre = pl.pallas_call(
    _pre_body,
    grid=(NTBL,),
    in_specs=[
        pl.BlockSpec((NCLU, H), lambda j: (0, 0)),
        pl.BlockSpec((NCELL, H), lambda j: (0, 0)),
        pl.BlockSpec((1, H, H), lambda j: (j, 0, 0)),
    ],
    out_specs=pl.BlockSpec((TBL, H), lambda j: (j, 0)),
    out_shape=jax.ShapeDtypeStruct((NTBL * TBL, H), F32),
)


def _mlp_body(ga_ref, gb_ref, w2_ref, w3_ref, w4_ref, bs_ref, out_ref):
    b1 = bs_ref[0, 0, :]
    b2 = bs_ref[0, 1, :]
    b3 = bs_ref[0, 2, :]
    b4 = bs_ref[0, 3, :]
    y = jnp.maximum(ga_ref[...] + gb_ref[...] + b1, 0.0).astype(BF16)
    y = jnp.maximum(jnp.dot(y, w2_ref[0], preferred_element_type=F32) + b2,
                    0.0).astype(BF16)
    y = jnp.maximum(jnp.dot(y, w3_ref[0], preferred_element_type=F32) + b3,
                    0.0).astype(BF16)
    out_ref[...] = jnp.dot(y, w4_ref[0], preferred_element_type=F32) + b4


def _mk_mlp(segs):
    ends, starts = segs['ends'], segs['starts']
    abase, bbase, types = segs['a'], segs['b'], segs['types']
    n = len(ends)

    def sel(i, vals):
        out = jnp.int32(vals[n - 1])
        for k in range(n - 2, -1, -1):
            out = jnp.where(i < ends[k], jnp.int32(vals[k]), out)
        return out

    def a_map(i):
        return sel(i, abase) + (i - sel(i, starts)), 0

    def b_map(i):
        return sel(i, bbase) + (i - sel(i, starts)), 0

    def t_map(i):
        return sel(i, types), 0, 0

    return pl.pallas_call(
        _mlp_body,
        grid=(segs['nblocks'],),
        in_specs=[
            pl.BlockSpec((BLK, H), a_map),
            pl.BlockSpec((BLK, H), b_map),
            pl.BlockSpec((1, H, H), t_map),
            pl.BlockSpec((1, H, H), t_map),
            pl.BlockSpec((1, H, H), t_map),
            pl.BlockSpec((1, 4, H), t_map),
        ],
        out_specs=pl.BlockSpec((BLK, H), lambda i: (i, 0)),
        out_shape=jax.ShapeDtypeStruct((segs['nm'], H), F32),
    )


_mlp1 = _mk_mlp(SEGS1)
_mlp2 = _mk_mlp(SEGS2)


def _lstm_factory(first):
    def body(cx_ref, pc_ref, lg_ref, p5_ref, wih_ref, whh_ref, hp_ref,
             cp_ref, h_ref, c_ref):
        parts = [cx_ref[...]]
        for t in range(2):
            parts.append(pc_ref[t, 0] + pc_ref[t, 1])
        parts.append(lg_ref[0])
        parts.append(lg_ref[1])
        parts.append(p5_ref[0] + p5_ref[1])
        x6 = jnp.concatenate(parts, axis=-1)
        gates = jnp.dot(x6, wih_ref[...], preferred_element_type=F32)
        if not first:
            gates = gates + jnp.dot(hp_ref[...], whh_ref[...],
                                    preferred_element_type=F32)
        i = jax.nn.sigmoid(gates[:, 0:H])
        f = jax.nn.sigmoid(gates[:, H:2 * H])
        g = jnp.tanh(gates[:, 2 * H:3 * H])
        o = jax.nn.sigmoid(gates[:, 3 * H:4 * H])
        cn = i * g if first else f * cp_ref[...] + i * g
        c_ref[...] = cn
        h_ref[...] = o * jnp.tanh(cn)

    nb = NCELL // BLK
    return pl.pallas_call(
        body,
        grid=(nb,),
        in_specs=[
            pl.BlockSpec((BLK, H), lambda b: (b, 0)),
            pl.BlockSpec((2, 2, BLK, H), lambda b: (0, 0, b, 0)),
            pl.BlockSpec((2, BLK, H), lambda b: (0, b, 0)),
            pl.BlockSpec((2, BLK, H), lambda b: (0, b, 0)),
            pl.BlockSpec((6 * H, 4 * H), lambda b: (0, 0)),
            pl.BlockSpec((H, 4 * H), lambda b: (0, 0)),
            pl.BlockSpec((BLK, H), lambda b: (b, 0)),
            pl.BlockSpec((BLK, H), lambda b: (b, 0)),
        ],
        out_specs=[
            pl.BlockSpec((BLK, H), lambda b: (b, 0)),
            pl.BlockSpec((BLK, H), lambda b: (b, 0)),
        ],
        out_shape=[
            jax.ShapeDtypeStruct((NCELL, H), F32),
            jax.ShapeDtypeStruct((NCELL, H), F32),
        ],
    )


_lstm_first = _lstm_factory(True)
_lstm_rest = _lstm_factory(False)


def _segltgt_body(m_ref, dlt_ref, dgt_ref, out_ref):
    b = pl.program_id(0)
    rows = lax.broadcasted_iota(jnp.int32, (BLK, E_LT), 0) + b * BLK
    m_lt = m_ref[0:E_LT, :].astype(BF16)
    m_gt = m_ref[E_LT:2 * E_LT, :].astype(BF16)
    oh_lt = (rows == dlt_ref[...]).astype(BF16)
    oh_gt = (rows == dgt_ref[...]).astype(BF16)
    out_ref[0] = jnp.dot(oh_lt, m_lt, preferred_element_type=F32)
    out_ref[1] = jnp.dot(oh_gt, m_gt, preferred_element_type=F32)


_segltgt = pl.pallas_call(
    _segltgt_body,
    grid=(NCELL // BLK,),
    in_specs=[
        pl.BlockSpec((2 * E_LT, H), lambda b: (MO_LT // (2 * E_LT), 0)),
        pl.BlockSpec((1, E_LT), lambda b: (0, 0)),
        pl.BlockSpec((1, E_LT), lambda b: (0, 0)),
    ],
    out_specs=pl.BlockSpec((2, BLK, H), lambda b: (0, b, 0)),
    out_shape=jax.ShapeDtypeStruct((2, NCELL, H), F32),
)

NPAD = 16384


def _logits_body(h0_ref, h1_ref, h2_ref, h3_ref, oe_ref, out_ref):
    hs = [h0_ref, h1_ref, h2_ref, h3_ref]
    for s in range(4):
        hv = hs[s][...]
        for b in range(8):
            x = hv[128 * b:128 * (b + 1), :]
            out_ref[s, 128 * b:128 * (b + 1), :] = jnp.dot(
                x, oe_ref[b], preferred_element_type=F32)


_logits = pl.pallas_call(
    _logits_body,
    grid=(16,),
    in_specs=[pl.BlockSpec((1024, H), lambda g: (g, 0))] * 4 + [
        pl.BlockSpec((8, H, 16), lambda g: (g, 0, 0)),
    ],
    out_specs=pl.BlockSpec((4, 1024, 16), lambda g: (0, g, 0)),
    out_shape=jax.ShapeDtypeStruct((4, NPAD, 16), F32),
)


def _mk_gather_body(gch):
    nrounds = gch // GB_CH

    def body(tbl, idx, out, idx_v, buf0, buf1, gs0, gs1, ws0, ws1):
        c = lax.axis_index("c")
        s = lax.axis_index("s")
        wid = c * 16 + s
        pltpu.sync_copy(idx.at[wid], idx_v)
        base = wid * gch * 128

        def fire(r, buf, sem):
            for k in range(GB_CH):
                pltpu.async_copy(tbl.at[idx_v.at[r * GB_CH + k]],
                                 buf.at[pl.ds(k * 128, 128)], sem)

        def drain_gathers(buf, sem):
            pltpu.make_async_copy(tbl.at[pl.ds(0, GB_ROWS)], buf, sem).wait()

        def out_rows(r):
            return out.at[pl.ds(base + r * GB_ROWS, GB_ROWS)]

        fire(0, buf0, gs0)
        fire(1, buf1, gs1)

        def loop(it, _):
            r0 = it * 2
            r1 = r0 + 1
            drain_gathers(buf0, gs0)
            pltpu.async_copy(buf0, out_rows(r0), ws0)
            drain_gathers(buf1, gs1)
            pltpu.async_copy(buf1, out_rows(r1), ws1)

            @pl.when(it < nrounds // 2 - 1)
            def _():
                pltpu.make_async_copy(buf0, out_rows(r0), ws0).wait()
                fire(r0 + 2, buf0, gs0)
                pltpu.make_async_copy(buf1, out_rows(r1), ws1).wait()
                fire(r1 + 2, buf1, gs1)
            return 0

        lax.fori_loop(0, nrounds // 2, loop, 0)
        last = nrounds - 1
        pltpu.make_async_copy(buf0, out_rows(last - 1), ws0).wait()
        pltpu.make_async_copy(buf1, out_rows(last), ws1).wait()

    return body


@functools.cache
def _sc_gather_kernel(gch, nrows):
    mesh = plsc.VectorSubcoreMesh(core_axis_name="c", subcore_axis_name="s")
    return pl.kernel(
        _mk_gather_body(gch),
        out_type=jax.ShapeDtypeStruct((nrows, H), F32),
        mesh=mesh,
        scratch_types=[
            pltpu.VMEM((gch, 128), jnp.int32),
            pltpu.VMEM((GB_ROWS, H), F32),
            pltpu.VMEM((GB_ROWS, H), F32),
            pltpu.SemaphoreType.DMA,
            pltpu.SemaphoreType.DMA,
            pltpu.SemaphoreType.DMA,
            pltpu.SemaphoreType.DMA,
        ],
    )


def _sc_gather1(tbl, idx):
    return _sc_gather_kernel(GCH1, NGP1)(tbl, idx)


def _sc_gather2(tbl, idx):
    return _sc_gather_kernel(GCH2, NGP2)(tbl, idx)


CSZ = 80


def _scatter_cell_phase(m, shc, ix2, mb0, mb1, sm0, sm1, s, wid,
                        pcell_view, zz, ix, nch, mbase):
    tb = mbase + wid * nch * CSZ
    ixw = ix.at[wid]
    bufs = (mb0, mb1)
    sems = (sm0, sm1)

    def issue(j, b):
        pltpu.async_copy(m.at[pl.ds(tb + j * CSZ, CSZ)], bufs[b], sems[b])
        pltpu.async_copy(ixw.at[j], ix2.at[b], sems[b])

    def wait_add(j, b):
        pltpu.make_async_copy(m.at[pl.ds(tb + j * CSZ, CSZ)],
                              bufs[b], sems[b]).wait()
        pltpu.make_async_copy(ixw.at[j], ix2.at[b], sems[b]).wait()
        pltpu.sync_copy(bufs[b], shc.at[ix2.at[b]], add=True)

    issue(0, 0)
    if nch > 1:
        issue(1, 1)
    plsc.subcore_barrier()

    if nch <= 2:
        wait_add(0, 0)
        if nch == 2:
            wait_add(1, 1)
    else:
        def loop(it, carry):
            j0 = it * 2
            j1 = j0 + 1
            wait_add(j0, 0)

            @pl.when(j0 + 2 < nch)
            def _():
                issue(j0 + 2, 0)

            @pl.when(j1 < nch)
            def _():
                wait_add(j1, 1)

                @pl.when(j1 + 2 < nch)
                def _():
                    issue(j1 + 2, 1)
            return carry

        lax.fori_loop(0, (nch + 1) // 2, loop, 0)
    plsc.subcore_barrier()
    for k in range(4):
        rows = pl.ds(s * 800 + k * 200, 200)
        pltpu.sync_copy(shc.at[rows], pcell_view.at[rows])
        pltpu.sync_copy(zz, shc.at[rows])


def _sc_scatter1_body(m, ixcon, ixmay, zz,
                      pcell, shc, ix2, mb0, mb1, sm0, sm1):
    c = lax.axis_index("c")
    s = lax.axis_index("s")
    wid = c * 16 + s
    for k in range(4):
        pltpu.sync_copy(zz, shc.at[pl.ds(s * 800 + k * 200, 200)])

    args = (m, shc, ix2, mb0, mb1, sm0, sm1, s, wid)
    _scatter_cell_phase(*args, pcell.at[0].at[c], zz, ixcon, 5, MO_CON)
    _scatter_cell_phase(*args, pcell.at[1].at[c], zz, ixmay, 55, MO_MAY)


def _sc_scatter2_body(m, ixdiff, ixclu, zz,
                      pcell, pclu, shc, ix2, mb0, mb1, sm0, sm1):
    c = lax.axis_index("c")
    s = lax.axis_index("s")
    wid = c * 16 + s
    for k in range(4):
        pltpu.sync_copy(zz, shc.at[pl.ds(s * 800 + k * 200, 200)])

    args = (m, shc, ix2, mb0, mb1, sm0, sm1, s, wid)
    _scatter_cell_phase(*args, pcell.at[c], zz, ixdiff, 90, MO_DIFF)

    pltpu.sync_copy(ixclu.at[wid], ix2.at[pl.ds(0, 1)])
    base = jnp.where(wid < 16, MO_CLT + wid * CSZ, MO_CGT + (wid - 16) * CSZ)
    plsc.subcore_barrier()
    pltpu.sync_copy(m.at[pl.ds(base, CSZ)], mb0)
    pltpu.sync_copy(mb0, shc.at[ix2.at[0]], add=True)
    plsc.subcore_barrier()
    rows = pl.ds(s * 88, 88)
    pltpu.sync_copy(shc.at[rows], pclu.at[c].at[rows])


def _sc_scratch():
    return [
        pltpu.VMEM_SHARED((NCELL, H), F32),
        pltpu.VMEM((2, CSZ), jnp.int32),
        pltpu.VMEM((CSZ, H), F32),
        pltpu.VMEM((CSZ, H), F32),
        pltpu.SemaphoreType.DMA,
        pltpu.SemaphoreType.DMA,
    ]


@functools.cache
def _sc_scatter1_kernel():
    mesh = plsc.VectorSubcoreMesh(core_axis_name="c", subcore_axis_name="s")
    return pl.kernel(
        _sc_scatter1_body,
        out_type=jax.ShapeDtypeStruct((2, 2, NCELL, H), F32),
        mesh=mesh,
        scratch_types=_sc_scratch(),
    )


@functools.cache
def _sc_scatter2_kernel():
    mesh = plsc.VectorSubcoreMesh(core_axis_name="c", subcore_axis_name="s")
    return pl.kernel(
        _sc_scatter2_body,
        out_type=(
            jax.ShapeDtypeStruct((2, NCELL, H), F32),
            jax.ShapeDtypeStruct((2, NCLU, H), F32),
        ),
        mesh=mesh,
        scratch_types=_sc_scratch(),
    )


def _sc_scatter1(m, ixcon, ixmay, zz):
    return _sc_scatter1_kernel()(m, ixcon, ixmay, zz)


def _sc_scatter2(m, ixdiff, ixclu, zz):
    return _sc_scatter2_kernel()(m, ixdiff, ixclu, zz)


def kernel(cell_x, cluster_x, output_embeddings, params, contains_src,
           contains_dst, may_src, may_dst, lt_edges, gt_edges, diff_edges,
           clt_edges, cgt_edges):
    i32 = jnp.int32
    gidx1 = jnp.concatenate([
        contains_src.astype(i32) + S_A_CON * TBL,
        contains_dst.astype(i32) + S_B_CON * TBL,
        may_src.astype(i32) + S_A_MAY * TBL,
        may_dst.astype(i32) + S_B_MAY * TBL,
        lt_edges[0].astype(i32) + S_A_LT * TBL,
        lt_edges[1].astype(i32) + S_B_LT * TBL,
        gt_edges[0].astype(i32) + S_A_GT * TBL,
        gt_edges[1].astype(i32) + S_B_GT * TBL,
        jnp.zeros((NGP1 - NG1,), i32),
    ]).reshape(NW, GCH1, 128)
    gidx2 = jnp.concatenate([
        diff_edges[0].astype(i32) + S_A_DIFF * TBL,
        diff_edges[1].astype(i32) + S_B_DIFF * TBL,
        jnp.pad(clt_edges[0].astype(i32) + S_A_CLT * TBL, (0, 256)),
        jnp.pad(clt_edges[1].astype(i32) + S_B_CLT * TBL, (0, 256)),
        jnp.pad(cgt_edges[0].astype(i32) + S_A_CGT * TBL, (0, 256)),
        jnp.pad(cgt_edges[1].astype(i32) + S_B_CGT * TBL, (0, 256)),
    ]).reshape(NW, GCH2, 128)

    ixcon = contains_dst.astype(i32).reshape(NW, 5, CSZ)
    ixmay = may_dst.astype(i32).reshape(NW, 55, CSZ)
    dlt = lt_edges[1].astype(i32).reshape(1, E_LT)
    dgt = gt_edges[1].astype(i32).reshape(1, E_LT)
    ixdiff = diff_edges[1].astype(i32).reshape(NW, 90, CSZ)
    ixclu = jnp.concatenate(
        [clt_edges[1].astype(i32), cgt_edges[1].astype(i32)]
    ).reshape(NW, 1, CSZ)
    zz = jnp.zeros((200, H), F32)

    order = ['contains', 'may_contain', 'lt', 'gt', 'diff']
    w1a = {t: params[t]['W1'][:H] for t in order}
    w1b = {t: params[t]['W1'][H:] for t in order}
    w14 = jnp.stack([
        w1a['contains'], w1a['may_contain'],
        w1a['lt'], w1b['lt'], w1a['gt'], w1b['gt'],
        w1a['lt'], w1a['gt'], w1a['diff'],
        w1b['contains'], w1b['may_contain'], w1b['lt'], w1b['gt'], w1b['diff'],
    ])
    w2s = jnp.stack([params[t]['W2'] for t in order]).astype(BF16)
    w3s = jnp.stack([params[t]['W3'] for t in order]).astype(BF16)
    w4s = jnp.stack([params[t]['W4'] for t in order]).astype(BF16)
    bs = jnp.stack([
        jnp.stack([params[t]['b1'], params[t]['b2'],
                   params[t]['b3'], params[t]['b4']]) for t in order])
    wihT = params['Wih'].T
    whhT = params['Whh'].T

    oeT = jnp.pad(
        output_embeddings.reshape(128, 11, H).transpose(0, 2, 1),
        ((0, 0), (0, 0), (0, 5)))

    h_cell = cell_x
    h_clu = cluster_x
    rnn_h = rnn_c = None
    p_clu = None
    outs = []
    for step in range(STEPS):
        if step:
            h_clu = _clup(h_clu, p_clu)
        tblv = _pre(h_clu, h_cell, w14)
        g1 = _sc_gather1(tblv, gidx1)
        m1 = _mlp1(g1, g1, w2s, w3s, w4s, bs)
        g2 = _sc_gather2(tblv, gidx2)
        m2 = _mlp2(g2, g2, w2s, w3s, w4s, bs)
        pc2 = _sc_scatter1(m1, ixcon, ixmay, zz)
        ltgt = _segltgt(m1, dlt, dgt)
        pc5, p_clu = _sc_scatter2(m2, ixdiff, ixclu, zz)
        if step == 0:
            zeros = jnp.zeros((NCELL, H), F32)
            rnn_h, rnn_c = _lstm_first(cell_x, pc2, ltgt, pc5, wihT, whhT,
                                       zeros, zeros)
        else:
            rnn_h, rnn_c = _lstm_rest(cell_x, pc2, ltgt, pc5, wihT, whhT,
                                      rnn_h, rnn_c)
        h_cell = rnn_h
        outs.append(rnn_h)

    hp = [
        jnp.pad(h.reshape(128, 100, H), ((0, 0), (0, 28), (0, 0)))
        .reshape(NPAD, H) for h in outs
    ]
    lg = _logits(hp[0], hp[1], hp[2], hp[3], oeT)
    return lg.reshape(4, 128, 128, 16)[:, :, :100, :11].reshape(4, NCELL, 11)
